# Initial kernel scaffold; baseline (speedup 1.0000x reference)
import functools

import jax
import jax.numpy as jnp
from jax import lax
from jax.experimental import pallas as pl
from jax.experimental.pallas import tpu as pltpu
from jax.experimental.pallas import tpu_sc as plsc

D = 64
NC = 2
NS = 16
NW = NC * NS
CHUNK = 128


def _sc_gather(table, idx):
    r = idx.shape[0]
    assert r % (NW * CHUNK) == 0
    per_w = r // NW
    n_chunks = per_w // CHUNK
    idx3 = idx.reshape(NW, n_chunks, CHUNK)
    mesh = plsc.VectorSubcoreMesh(core_axis_name="c", subcore_axis_name="s")

    @functools.partial(
        pl.kernel,
        out_type=jax.ShapeDtypeStruct((r, D), jnp.float32),
        mesh=mesh,
        scratch_types=[
            pltpu.VMEM((n_chunks, CHUNK), jnp.int32),
            pltpu.VMEM((CHUNK, D), jnp.float32),
            pltpu.SemaphoreType.DMA,
        ],
    )
    def gather_kernel(table_hbm, idx_hbm, out_hbm, idx_v, rows_v, gsem):
        wid = lax.axis_index("s") * NC + lax.axis_index("c")
        base = wid * per_w
        pltpu.sync_copy(idx_hbm.at[wid], idx_v)

        def body(j, carry):
            pltpu.async_copy(table_hbm.at[idx_v.at[j]], rows_v, gsem).wait()
            pltpu.sync_copy(rows_v, out_hbm.at[pl.ds(base + j * CHUNK, CHUNK)])
            return carry

        lax.fori_loop(0, n_chunks, body, 0)

    return gather_kernel(table, idx3)


def _tc_project(y, role_flat, w, b2):
    n = y.shape[0]
    kd = y.shape[1]
    blk = 1024
    assert n % blk == 0

    def body(y_ref, r_ref, w_ref, b_ref, o_ref):
        yy = y_ref[...] + r_ref[...]
        o_ref[...] = (
            jnp.dot(yy, w_ref[...], preferred_element_type=jnp.float32)
            + b_ref[...]
        )

    return pl.pallas_call(
        body,
        grid=(n // blk,),
        in_specs=[
            pl.BlockSpec((blk, kd), lambda i: (i, 0)),
            pl.BlockSpec((1, kd), lambda i: (0, 0)),
            pl.BlockSpec((kd, D), lambda i: (0, 0)),
            pl.BlockSpec((1, D), lambda i: (0, 0)),
        ],
        out_specs=pl.BlockSpec((blk, D), lambda i: (i, 0)),
        out_shape=jax.ShapeDtypeStruct((n, D), jnp.float32),
    )(y, role_flat, w, b2)


def kernel(x, emb_table, role_table, W, b):
    bb, m, kk = x.shape
    idx_flat = x.reshape(-1).astype(jnp.int32)
    rows = _sc_gather(emb_table, idx_flat)
    y = rows.reshape(bb * m, kk * D)
    out = _tc_project(y, role_table.reshape(1, kk * D), W, b.reshape(1, D))
    return out.reshape(bb, m, D)


# speedup vs baseline: 1.1176x; 1.1176x over previous
import functools

import jax
import jax.numpy as jnp
from jax import lax
from jax.experimental import pallas as pl
from jax.experimental.pallas import tpu as pltpu
from jax.experimental.pallas import tpu_sc as plsc

D = 64
NC = 2
NS = 16
NW = NC * NS
CHUNK = 128


def _sc_gather(table, idx):
    r = idx.shape[0]
    assert r % (NW * CHUNK) == 0
    per_w = r // NW
    n_chunks = per_w // CHUNK
    idx3 = idx.reshape(NW, n_chunks, CHUNK)
    mesh = plsc.VectorSubcoreMesh(core_axis_name="c", subcore_axis_name="s")

    @functools.partial(
        pl.kernel,
        out_type=jax.ShapeDtypeStruct((r, D), jnp.float32),
        mesh=mesh,
        scratch_types=[
            pltpu.VMEM((n_chunks, CHUNK), jnp.int32),
            pltpu.VMEM((CHUNK, D), jnp.float32),
            pltpu.SemaphoreType.DMA,
        ],
        compiler_params=pltpu.CompilerParams(use_tc_tiling_on_sc=False),
    )
    def gather_kernel(table_hbm, idx_hbm, out_hbm, idx_v, rows_v, gsem):
        wid = lax.axis_index("s") * NC + lax.axis_index("c")
        base = wid * per_w
        pltpu.sync_copy(idx_hbm.at[wid], idx_v)

        def body(j, carry):
            pltpu.async_copy(table_hbm.at[idx_v.at[j]], rows_v, gsem).wait()
            pltpu.sync_copy(rows_v, out_hbm.at[pl.ds(base + j * CHUNK, CHUNK)])
            return carry

        lax.fori_loop(0, n_chunks, body, 0)

    return gather_kernel(table, idx3)


def _tc_project(y, role_flat, w, b2):
    n = y.shape[0]
    kd = y.shape[1]
    blk = 1024
    assert n % blk == 0

    def body(y_ref, r_ref, w_ref, b_ref, o_ref):
        yy = y_ref[...] + r_ref[...]
        o_ref[...] = (
            jnp.dot(yy, w_ref[...], preferred_element_type=jnp.float32)
            + b_ref[...]
        )

    return pl.pallas_call(
        body,
        grid=(n // blk,),
        in_specs=[
            pl.BlockSpec((blk, kd), lambda i: (i, 0)),
            pl.BlockSpec((1, kd), lambda i: (0, 0)),
            pl.BlockSpec((kd, D), lambda i: (0, 0)),
            pl.BlockSpec((1, D), lambda i: (0, 0)),
        ],
        out_specs=pl.BlockSpec((blk, D), lambda i: (i, 0)),
        out_shape=jax.ShapeDtypeStruct((n, D), jnp.float32),
    )(y, role_flat, w, b2)


def kernel(x, emb_table, role_table, W, b):
    bb, m, kk = x.shape
    idx_flat = x.reshape(-1).astype(jnp.int32)
    rows = _sc_gather(emb_table, idx_flat)
    y = rows.reshape(bb * m, kk * D)
    out = _tc_project(y, role_table.reshape(1, kk * D), W, b.reshape(1, D))
    return out.reshape(bb, m, D)


# speedup vs baseline: 1.2117x; 1.0842x over previous
import functools

import jax
import jax.numpy as jnp
from jax import lax
from jax.experimental import pallas as pl
from jax.experimental.pallas import tpu as pltpu
from jax.experimental.pallas import tpu_sc as plsc

D = 64
NC = 2
NS = 16
NW = NC * NS
CHUNK = 128


SUPER = 2
NBUF = 3


def _sc_gather(table, idx):
    r = idx.shape[0]
    rows_per_buf = SUPER * CHUNK
    assert r % (NW * rows_per_buf * NBUF) == 0
    per_w = r // NW
    n_chunks = per_w // CHUNK
    n_super = per_w // rows_per_buf
    n_bodies = n_super // NBUF
    idx3 = idx.reshape(NW, n_chunks, CHUNK)
    mesh = plsc.VectorSubcoreMesh(core_axis_name="c", subcore_axis_name="s")

    @functools.partial(
        pl.kernel,
        out_type=jax.ShapeDtypeStruct((r, D), jnp.float32),
        mesh=mesh,
        scratch_types=[
            pltpu.VMEM((n_chunks, CHUNK), jnp.int32),
            [pltpu.VMEM((rows_per_buf, D), jnp.float32) for _ in range(NBUF)],
            [pltpu.SemaphoreType.DMA for _ in range(NBUF)],
            [pltpu.SemaphoreType.DMA for _ in range(NBUF)],
        ],
        compiler_params=pltpu.CompilerParams(use_tc_tiling_on_sc=False),
    )
    def gather_kernel(table_hbm, idx_hbm, out_hbm, idx_v, bufs, gsems, wsems):
        wid = lax.axis_index("s") * NC + lax.axis_index("c")
        base = wid * per_w
        pltpu.sync_copy(idx_hbm.at[wid], idx_v)

        def fire_gathers(s, i):
            for u in range(SUPER):
                pltpu.make_async_copy(
                    table_hbm.at[idx_v.at[s * SUPER + u]],
                    bufs[i].at[pl.ds(u * CHUNK, CHUNK)],
                    gsems[i],
                ).start()

        def wait_gathers(s, i):
            for u in range(SUPER):
                pltpu.make_async_copy(
                    table_hbm.at[idx_v.at[s * SUPER + u]],
                    bufs[i].at[pl.ds(u * CHUNK, CHUNK)],
                    gsems[i],
                ).wait()

        def write_copy(s, i):
            return pltpu.make_async_copy(
                bufs[i],
                out_hbm.at[pl.ds(base + s * rows_per_buf, rows_per_buf)],
                wsems[i],
            )

        for i in range(NBUF):
            fire_gathers(i, i)

        def body(t, carry):
            s0 = t * NBUF
            for i in range(NBUF):
                wait_gathers(s0 + i, i)
                write_copy(s0 + i, i).start()
            for i in range(NBUF):
                write_copy(s0 + i, i).wait()

                @pl.when(t + 1 < n_bodies)
                def _():
                    fire_gathers(s0 + NBUF + i, i)

            return carry

        lax.fori_loop(0, n_bodies, body, 0)

    return gather_kernel(table, idx3)


def _tc_project(y, role_flat, w, b2):
    n = y.shape[0]
    kd = y.shape[1]
    blk = 1024
    assert n % blk == 0

    def body(y_ref, r_ref, w_ref, b_ref, o_ref):
        yy = y_ref[...] + r_ref[...]
        o_ref[...] = (
            jnp.dot(yy, w_ref[...], preferred_element_type=jnp.float32)
            + b_ref[...]
        )

    return pl.pallas_call(
        body,
        grid=(n // blk,),
        in_specs=[
            pl.BlockSpec((blk, kd), lambda i: (i, 0)),
            pl.BlockSpec((1, kd), lambda i: (0, 0)),
            pl.BlockSpec((kd, D), lambda i: (0, 0)),
            pl.BlockSpec((1, D), lambda i: (0, 0)),
        ],
        out_specs=pl.BlockSpec((blk, D), lambda i: (i, 0)),
        out_shape=jax.ShapeDtypeStruct((n, D), jnp.float32),
    )(y, role_flat, w, b2)


def kernel(x, emb_table, role_table, W, b):
    bb, m, kk = x.shape
    idx_flat = x.reshape(-1).astype(jnp.int32)
    rows = _sc_gather(emb_table, idx_flat)
    y = rows.reshape(bb * m, kk * D)
    out = _tc_project(y, role_table.reshape(1, kk * D), W, b.reshape(1, D))
    return out.reshape(bb, m, D)


# speedup vs baseline: 1.2397x; 1.0232x over previous
import functools

import jax
import jax.numpy as jnp
from jax import lax
from jax.experimental import pallas as pl
from jax.experimental.pallas import tpu as pltpu
from jax.experimental.pallas import tpu_sc as plsc

D = 64
NC = 2
NS = 16
NW = NC * NS
CHUNK = 128


SUPER = 2
NBUF = 3


def _sc_gather(table, idx):
    r = idx.shape[0]
    rows_per_buf = SUPER * CHUNK
    assert r % (NW * rows_per_buf * NBUF) == 0
    per_w = r // NW
    n_chunks = per_w // CHUNK
    n_super = per_w // rows_per_buf
    n_bodies = n_super // NBUF
    idx3 = idx.reshape(NW, n_chunks, CHUNK)
    mesh = plsc.VectorSubcoreMesh(core_axis_name="c", subcore_axis_name="s")

    @functools.partial(
        pl.kernel,
        out_type=jax.ShapeDtypeStruct((r, D), jnp.float32),
        mesh=mesh,
        scratch_types=[
            pltpu.VMEM((n_chunks, CHUNK), jnp.int32),
            [pltpu.VMEM((rows_per_buf, D), jnp.float32) for _ in range(NBUF)],
            [pltpu.SemaphoreType.DMA for _ in range(NBUF)],
            [pltpu.SemaphoreType.DMA for _ in range(NBUF)],
        ],
        compiler_params=pltpu.CompilerParams(use_tc_tiling_on_sc=False),
    )
    def gather_kernel(table_hbm, idx_hbm, out_hbm, idx_v, bufs, gsems, wsems):
        wid = lax.axis_index("s") * NC + lax.axis_index("c")
        base = wid * per_w
        pltpu.sync_copy(idx_hbm.at[wid], idx_v)

        def fire_gathers(s, i):
            for u in range(SUPER):
                pltpu.make_async_copy(
                    table_hbm.at[idx_v.at[s * SUPER + u]],
                    bufs[i].at[pl.ds(u * CHUNK, CHUNK)],
                    gsems[i],
                ).start()

        def wait_gathers(s, i):
            for u in range(SUPER):
                pltpu.make_async_copy(
                    table_hbm.at[idx_v.at[s * SUPER + u]],
                    bufs[i].at[pl.ds(u * CHUNK, CHUNK)],
                    gsems[i],
                ).wait()

        def write_copy(s, i):
            return pltpu.make_async_copy(
                bufs[i],
                out_hbm.at[pl.ds(base + s * rows_per_buf, rows_per_buf)],
                wsems[i],
            )

        for i in range(NBUF):
            fire_gathers(i, i)

        def body(t, carry):
            s0 = t * NBUF
            for i in range(NBUF):
                wait_gathers(s0 + i, i)
                write_copy(s0 + i, i).start()
            for i in range(NBUF):
                write_copy(s0 + i, i).wait()

                @pl.when(t + 1 < n_bodies)
                def _():
                    fire_gathers(s0 + NBUF + i, i)

            return carry

        lax.fori_loop(0, n_bodies, body, 0)

    return gather_kernel(table, idx3)


def _tc_project(y4, w3, role, b2):
    kk, m, bb, d = y4.shape

    def body(y_ref, w_ref, r_ref, b_ref, o_ref):
        acc = b_ref[...]
        for k in range(kk):
            wk = w_ref[k]
            acc = acc + lax.dot_general(
                wk, r_ref[k], (((0,), (0,)), ((), ())),
                preferred_element_type=jnp.float32,
            )
        o = acc
        for k in range(kk):
            o = o + lax.dot_general(
                w_ref[k], y_ref[k, 0], (((0,), (1,)), ((), ())),
                preferred_element_type=jnp.float32,
            )
        o_ref[0] = o

    return pl.pallas_call(
        body,
        grid=(m,),
        in_specs=[
            pl.BlockSpec((kk, 1, bb, d), lambda i: (0, i, 0, 0)),
            pl.BlockSpec((kk, d, D), lambda i: (0, 0, 0)),
            pl.BlockSpec((kk, d, 1), lambda i: (0, 0, 0)),
            pl.BlockSpec((D, 1), lambda i: (0, 0)),
        ],
        out_specs=pl.BlockSpec((1, D, bb), lambda i: (i, 0, 0)),
        out_shape=jax.ShapeDtypeStruct((m, D, bb), jnp.float32),
    )(y4, w3, role, b2)


def kernel(x, emb_table, role_table, W, b):
    bb, m, kk = x.shape
    idx_flat = x.transpose(2, 1, 0).reshape(-1).astype(jnp.int32)
    rows = _sc_gather(emb_table, idx_flat)
    y4 = rows.reshape(kk, m, bb, D)
    out_phys = _tc_project(
        y4,
        W.reshape(kk, D, D),
        role_table.reshape(kk, D, 1),
        b.reshape(D, 1),
    )
    return out_phys.transpose(2, 0, 1)


# speedup vs baseline: 1.3697x; 1.1048x over previous
import functools

import jax
import jax.numpy as jnp
from jax import lax
from jax.experimental import pallas as pl
from jax.experimental.pallas import tpu as pltpu
from jax.experimental.pallas import tpu_sc as plsc

VOCAB = 1000000
D = 64
NC = 2
NS = 16
NW = NC * NS
CHUNK = 128


SUPER = 2
NBUF = 3


def _sc_gather(table, idx):
    r = idx.shape[0]
    rows_per_buf = SUPER * CHUNK
    assert r % (NW * rows_per_buf * NBUF) == 0
    per_w = r // NW
    n_chunks = per_w // CHUNK
    n_super = per_w // rows_per_buf
    n_bodies = n_super // NBUF
    idx3 = idx.reshape(NW, n_chunks, CHUNK)
    mesh = plsc.VectorSubcoreMesh(core_axis_name="c", subcore_axis_name="s")

    @functools.partial(
        pl.kernel,
        out_type=jax.ShapeDtypeStruct((r, D), jnp.float32),
        mesh=mesh,
        scratch_types=[
            pltpu.VMEM((n_chunks, CHUNK), jnp.int32),
            [pltpu.VMEM((rows_per_buf, D), jnp.float32) for _ in range(NBUF)],
            [pltpu.SemaphoreType.DMA for _ in range(NBUF)],
            [pltpu.SemaphoreType.DMA for _ in range(NBUF)],
        ],
        compiler_params=pltpu.CompilerParams(use_tc_tiling_on_sc=False),
    )
    def gather_kernel(table_hbm, idx_hbm, out_hbm, idx_v, bufs, gsems, wsems):
        wid = lax.axis_index("s") * NC + lax.axis_index("c")
        base = wid * per_w
        pltpu.sync_copy(idx_hbm.at[wid], idx_v)

        def fire_gathers(s, i):
            for u in range(SUPER):
                pltpu.make_async_copy(
                    table_hbm.at[idx_v.at[s * SUPER + u]],
                    bufs[i].at[pl.ds(u * CHUNK, CHUNK)],
                    gsems[i],
                ).start()

        def wait_gathers(s, i):
            for u in range(SUPER):
                pltpu.make_async_copy(
                    table_hbm.at[idx_v.at[s * SUPER + u]],
                    bufs[i].at[pl.ds(u * CHUNK, CHUNK)],
                    gsems[i],
                ).wait()

        def write_copy(s, i):
            return pltpu.make_async_copy(
                bufs[i],
                out_hbm.at[pl.ds(base + s * rows_per_buf, rows_per_buf)],
                wsems[i],
            )

        for i in range(NBUF):
            fire_gathers(i, i)

        def body(t, carry):
            s0 = t * NBUF
            for i in range(NBUF):
                wait_gathers(s0 + i, i)
                write_copy(s0 + i, i).start()
            for i in range(NBUF):
                write_copy(s0 + i, i).wait()

                @pl.when(t + 1 < n_bodies)
                def _():
                    fire_gathers(s0 + NBUF + i, i)

            return carry

        lax.fori_loop(0, n_bodies, body, 0)

    return gather_kernel(table, idx3)


TBLK = 2048


def _tc_transpose_table(emb_t, vocab):
    grid = pl.cdiv(vocab, TBLK)

    def body(x_ref, o_ref):
        x = x_ref[...]
        r = lax.broadcasted_iota(jnp.int32, (D, 2 * D), 0)
        c = lax.broadcasted_iota(jnp.int32, (D, 2 * D), 1)
        eye2 = jnp.where(r == c % D, 1.0, 0.0).astype(jnp.float32)
        o_ref[...] = lax.dot_general(
            x, eye2, (((0,), (0,)), ((), ())),
            preferred_element_type=jnp.float32,
        )

    return pl.pallas_call(
        body,
        grid=(grid,),
        in_specs=[pl.BlockSpec((D, TBLK), lambda i: (0, i))],
        out_specs=pl.BlockSpec((TBLK, 2 * D), lambda i: (i, 0)),
        out_shape=jax.ShapeDtypeStruct((vocab, 2 * D), jnp.float32),
    )(emb_t)


def _tc_project(y4, w3, role, b2):
    kk, m, bb, d = y4.shape

    def body(y_ref, w_ref, r_ref, b_ref, o_ref):
        acc = b_ref[...]
        for k in range(kk):
            wk = w_ref[k]
            acc = acc + lax.dot_general(
                wk, r_ref[k], (((0,), (0,)), ((), ())),
                preferred_element_type=jnp.float32,
            )
        o = acc
        for k in range(kk):
            o = o + lax.dot_general(
                w_ref[k], y_ref[k, 0], (((0,), (1,)), ((), ())),
                preferred_element_type=jnp.float32,
            )
        o_ref[0] = o

    return pl.pallas_call(
        body,
        grid=(m,),
        in_specs=[
            pl.BlockSpec((kk, 1, bb, d), lambda i: (0, i, 0, 0)),
            pl.BlockSpec((kk, d, D), lambda i: (0, 0, 0)),
            pl.BlockSpec((kk, d, 1), lambda i: (0, 0, 0)),
            pl.BlockSpec((D, 1), lambda i: (0, 0)),
        ],
        out_specs=pl.BlockSpec((1, D, bb), lambda i: (i, 0, 0)),
        out_shape=jax.ShapeDtypeStruct((m, D, bb), jnp.float32),
    )(y4, w3, role, b2)


def kernel(x, emb_table, role_table, W, b):
    bb, m, kk = x.shape
    idx_flat = (x.transpose(2, 1, 0).reshape(-1) * 2).astype(jnp.int32)
    vocab = emb_table.shape[0]
    t128 = _tc_transpose_table(emb_table.T, vocab)
    table_lin = t128.reshape(2 * vocab, D)
    rows = _sc_gather(table_lin, idx_flat)
    y4 = rows.reshape(kk, m, bb, D)
    out_phys = _tc_project(
        y4,
        W.reshape(kk, D, D),
        role_table.reshape(kk, D, 1),
        b.reshape(D, 1),
    )
    return out_phys.transpose(2, 0, 1)


# speedup vs baseline: 1.5025x; 1.0970x over previous
import functools

import jax
import jax.numpy as jnp
from jax import lax
from jax.experimental import pallas as pl
from jax.experimental.pallas import tpu as pltpu
from jax.experimental.pallas import tpu_sc as plsc

VOCAB = 1000000
D = 64
NC = 2
NS = 16
NW = NC * NS
CHUNK = 128


SUPER = 2
NBUF = 3


def _sc_gather(table, idx):
    r = idx.shape[0]
    rows_per_buf = SUPER * CHUNK
    assert r % (NW * rows_per_buf * NBUF) == 0
    per_w = r // NW
    n_chunks = per_w // CHUNK
    n_super = per_w // rows_per_buf
    n_bodies = n_super // NBUF
    idx3 = idx.reshape(NW, n_chunks, CHUNK)
    mesh = plsc.VectorSubcoreMesh(core_axis_name="c", subcore_axis_name="s")

    @functools.partial(
        pl.kernel,
        out_type=jax.ShapeDtypeStruct((r, D), jnp.float32),
        mesh=mesh,
        scratch_types=[
            pltpu.VMEM((n_chunks, CHUNK), jnp.int32),
            [pltpu.VMEM((rows_per_buf, D), jnp.float32) for _ in range(NBUF)],
            [pltpu.SemaphoreType.DMA for _ in range(NBUF)],
            [pltpu.SemaphoreType.DMA for _ in range(NBUF)],
        ],
        compiler_params=pltpu.CompilerParams(use_tc_tiling_on_sc=False),
    )
    def gather_kernel(table_hbm, idx_hbm, out_hbm, idx_v, bufs, gsems, wsems):
        wid = lax.axis_index("s") * NC + lax.axis_index("c")
        base = wid * per_w
        pltpu.sync_copy(idx_hbm.at[wid], idx_v)

        def fire_gathers(s, i):
            for u in range(SUPER):
                pltpu.make_async_copy(
                    table_hbm.at[idx_v.at[s * SUPER + u]],
                    bufs[i].at[pl.ds(u * CHUNK, CHUNK)],
                    gsems[i],
                ).start()

        def wait_gathers(s, i):
            for u in range(SUPER):
                pltpu.make_async_copy(
                    table_hbm.at[idx_v.at[s * SUPER + u]],
                    bufs[i].at[pl.ds(u * CHUNK, CHUNK)],
                    gsems[i],
                ).wait()

        def write_copy(s, i):
            return pltpu.make_async_copy(
                bufs[i],
                out_hbm.at[pl.ds(base + s * rows_per_buf, rows_per_buf)],
                wsems[i],
            )

        for i in range(NBUF):
            fire_gathers(i, i)

        def body(t, carry):
            s0 = t * NBUF
            for i in range(NBUF):
                wait_gathers(s0 + i, i)
                write_copy(s0 + i, i).start()
            for i in range(NBUF):
                write_copy(s0 + i, i).wait()

                @pl.when(t + 1 < n_bodies)
                def _():
                    fire_gathers(s0 + NBUF + i, i)

            return carry

        lax.fori_loop(0, n_bodies, body, 0)

    return gather_kernel(table, idx3)


TBLK = 2048


def _tc_transpose_table(emb_t, vocab):
    grid = pl.cdiv(vocab, TBLK)

    def body(x_ref, o_ref):
        x = x_ref[...]
        r = lax.broadcasted_iota(jnp.int32, (D, 2 * D), 0)
        c = lax.broadcasted_iota(jnp.int32, (D, 2 * D), 1)
        eye2 = jnp.where(r == c % D, 1.0, 0.0).astype(jnp.float32)
        o_ref[...] = lax.dot_general(
            x, eye2, (((0,), (0,)), ((), ())),
            preferred_element_type=jnp.float32,
        )

    return pl.pallas_call(
        body,
        grid=(grid,),
        in_specs=[pl.BlockSpec((D, TBLK), lambda i: (0, i))],
        out_specs=pl.BlockSpec((TBLK, 2 * D), lambda i: (i, 0)),
        out_shape=jax.ShapeDtypeStruct((vocab, 2 * D), jnp.float32),
    )(emb_t)


def _tc_project(yp, w2, role2, b2):
    kk, m, bb2, d2 = yp.shape
    bb = 2 * bb2

    def body(y_ref, w_ref, r_ref, b_ref, o_ref):
        c = b_ref[...]
        for k in range(kk):
            c = c + lax.dot_general(
                r_ref[k], w_ref[k], (((1,), (0,)), ((), ())),
                preferred_element_type=jnp.float32,
            )
        o = jnp.broadcast_to(c, (bb2, d2))
        for k in range(kk):
            o = o + lax.dot_general(
                y_ref[k, 0], w_ref[k], (((1,), (0,)), ((), ())),
                preferred_element_type=jnp.float32,
            )
        t = o.T
        o_ref[0] = jnp.concatenate([t[:D, :], t[D:, :]], axis=1)

    return pl.pallas_call(
        body,
        grid=(m,),
        in_specs=[
            pl.BlockSpec((kk, 1, bb2, d2), lambda i: (0, i, 0, 0)),
            pl.BlockSpec((kk, d2, d2), lambda i: (0, 0, 0)),
            pl.BlockSpec((kk, 1, d2), lambda i: (0, 0, 0)),
            pl.BlockSpec((1, d2), lambda i: (0, 0)),
        ],
        out_specs=pl.BlockSpec((1, D, bb), lambda i: (i, 0, 0)),
        out_shape=jax.ShapeDtypeStruct((m, D, bb), jnp.float32),
    )(yp, w2, role2, b2)


def kernel(x, emb_table, role_table, W, b):
    bb, m, kk = x.shape
    xp = x.transpose(2, 1, 0).reshape(kk, m, 2, bb // 2).transpose(0, 1, 3, 2)
    idx_flat = (xp.reshape(-1) * 2).astype(jnp.int32)
    vocab = emb_table.shape[0]
    t128 = _tc_transpose_table(emb_table.T, vocab)
    table_lin = t128.reshape(2 * vocab, D)
    rows = _sc_gather(table_lin, idx_flat)
    yp = rows.reshape(kk, m, bb // 2, 2 * D)
    w3 = W.reshape(kk, D, D)
    z = jnp.zeros((kk, D, D), jnp.float32)
    w2 = jnp.concatenate(
        [
            jnp.concatenate([w3, z], axis=2),
            jnp.concatenate([z, w3], axis=2),
        ],
        axis=1,
    )
    role2 = jnp.tile(role_table.reshape(kk, 1, D), (1, 1, 2))
    b2 = jnp.tile(b.reshape(1, D), (1, 2))
    out_phys = _tc_project(yp, w2, role2, b2)
    return out_phys.transpose(2, 0, 1)


# speedup vs baseline: 2.0055x; 1.3348x over previous
import functools

import jax
import jax.numpy as jnp
from jax import lax
from jax.experimental import pallas as pl
from jax.experimental.pallas import tpu as pltpu
from jax.experimental.pallas import tpu_sc as plsc

VOCAB = 1000000
D = 64
NC = 2
NS = 16
NW = NC * NS
CHUNK = 128


SUPER = 2
NBUF = 3


def _sc_gather(table, idx):
    r = idx.shape[0]
    rows_per_buf = SUPER * CHUNK
    assert r % (NW * rows_per_buf * NBUF) == 0
    per_w = r // NW
    n_chunks = per_w // CHUNK
    n_super = per_w // rows_per_buf
    n_bodies = n_super // NBUF
    idx3 = idx.reshape(NW, n_chunks, CHUNK)
    mesh = plsc.VectorSubcoreMesh(core_axis_name="c", subcore_axis_name="s")

    @functools.partial(
        pl.kernel,
        out_type=jax.ShapeDtypeStruct((r, D), jnp.float32),
        mesh=mesh,
        scratch_types=[
            pltpu.VMEM((n_chunks, CHUNK), jnp.int32),
            [pltpu.VMEM((rows_per_buf, D), jnp.float32) for _ in range(NBUF)],
            [pltpu.SemaphoreType.DMA for _ in range(NBUF)],
            [pltpu.SemaphoreType.DMA for _ in range(NBUF)],
        ],
        compiler_params=pltpu.CompilerParams(use_tc_tiling_on_sc=False),
    )
    def gather_kernel(table_hbm, idx_hbm, out_hbm, idx_v, bufs, gsems, wsems):
        wid = lax.axis_index("s") * NC + lax.axis_index("c")
        base = wid * per_w
        pltpu.sync_copy(idx_hbm.at[wid], idx_v)

        def fire_gathers(s, i):
            for u in range(SUPER):
                pltpu.make_async_copy(
                    table_hbm.at[idx_v.at[s * SUPER + u]],
                    bufs[i].at[pl.ds(u * CHUNK, CHUNK)],
                    gsems[i],
                ).start()

        def wait_gathers(s, i):
            for u in range(SUPER):
                pltpu.make_async_copy(
                    table_hbm.at[idx_v.at[s * SUPER + u]],
                    bufs[i].at[pl.ds(u * CHUNK, CHUNK)],
                    gsems[i],
                ).wait()

        def write_copy(s, i):
            return pltpu.make_async_copy(
                bufs[i],
                out_hbm.at[pl.ds(base + s * rows_per_buf, rows_per_buf)],
                wsems[i],
            )

        for i in range(NBUF):
            fire_gathers(i, i)

        def body(t, carry):
            s0 = t * NBUF
            for i in range(NBUF):
                wait_gathers(s0 + i, i)
                write_copy(s0 + i, i).start()
            for i in range(NBUF):
                write_copy(s0 + i, i).wait()

                @pl.when(t + 1 < n_bodies)
                def _():
                    fire_gathers(s0 + NBUF + i, i)

            return carry

        lax.fori_loop(0, n_bodies, body, 0)

    return gather_kernel(table, idx3)


TBLK = 2048


def _tc_transpose_table(emb_t, vocab):
    grid = pl.cdiv(vocab, 2 * TBLK)
    last = pl.cdiv(vocab, TBLK) - 1

    def body(x1_ref, x2_ref, o_ref):
        o_ref[...] = jnp.concatenate(
            [x1_ref[...].T, x2_ref[...].T], axis=1
        )

    return pl.pallas_call(
        body,
        grid=(grid,),
        in_specs=[
            pl.BlockSpec((D, TBLK), lambda i: (0, i)),
            pl.BlockSpec((D, TBLK), lambda i: (0, jnp.minimum(i + grid, last))),
        ],
        out_specs=pl.BlockSpec((TBLK, 2 * D), lambda i: (i, 0)),
        out_shape=jax.ShapeDtypeStruct((grid * TBLK, 2 * D), jnp.float32),
    )(emb_t, emb_t)


def _tc_project(yp, w2, role2, b2):
    kk, m, bb2, d2 = yp.shape

    def body(y_ref, w_ref, r_ref, b_ref, o_ref):
        c = b_ref[...]
        for k in range(kk):
            c = c + lax.dot_general(
                r_ref[k], w_ref[k], (((1,), (0,)), ((), ())),
                preferred_element_type=jnp.float32,
            )
        o = jnp.broadcast_to(c, (bb2, d2))
        for k in range(kk):
            o = o + lax.dot_general(
                y_ref[k, 0], w_ref[k], (((1,), (0,)), ((), ())),
                preferred_element_type=jnp.float32,
            )
        o_ref[0] = o

    return pl.pallas_call(
        body,
        grid=(m,),
        in_specs=[
            pl.BlockSpec((kk, 1, bb2, d2), lambda i: (0, i, 0, 0)),
            pl.BlockSpec((kk, d2, d2), lambda i: (0, 0, 0)),
            pl.BlockSpec((kk, 1, d2), lambda i: (0, 0, 0)),
            pl.BlockSpec((1, d2), lambda i: (0, 0)),
        ],
        out_specs=pl.BlockSpec((1, bb2, d2), lambda i: (i, 0, 0)),
        out_shape=jax.ShapeDtypeStruct((m, bb2, d2), jnp.float32),
    )(yp, w2, role2, b2)


def kernel(x, emb_table, role_table, W, b):
    bb, m, kk = x.shape
    vocab = emb_table.shape[0]
    s = TBLK * ((vocab + 2 * TBLK - 1) // (2 * TBLK))
    xi = x.transpose(2, 1, 0).reshape(-1).astype(jnp.int32)
    idx_flat = jnp.where(xi < s, 2 * xi, 2 * (xi - s) + 1)
    t128 = _tc_transpose_table(emb_table.T, vocab)
    table_lin = t128.reshape(2 * t128.shape[0], D)
    rows = _sc_gather(table_lin, idx_flat)
    yp = rows.reshape(kk, m, bb // 2, 2 * D)
    w3 = W.reshape(kk, D, D)
    z = jnp.zeros((kk, D, D), jnp.float32)
    w2 = jnp.concatenate(
        [
            jnp.concatenate([w3, z], axis=2),
            jnp.concatenate([z, w3], axis=2),
        ],
        axis=1,
    )
    role2 = jnp.tile(role_table.reshape(kk, 1, D), (1, 1, 2))
    b2 = jnp.tile(b.reshape(1, D), (1, 2))
    out_p = _tc_project(yp, w2, role2, b2)
    return out_p.reshape(m, bb, D).transpose(1, 0, 2)


# speedup vs baseline: 2.5354x; 1.2642x over previous
import functools

import jax
import jax.numpy as jnp
from jax import lax
from jax.experimental import pallas as pl
from jax.experimental.pallas import tpu as pltpu
from jax.experimental.pallas import tpu_sc as plsc

VOCAB = 1000000
D = 64
NC = 2
NS = 16
NW = NC * NS
CHUNK = 128


SUPER = 2
NBUF = 3


def _sc_gather(table, idx):
    r = idx.shape[0]
    rows_per_buf = SUPER * CHUNK
    assert r % (NW * rows_per_buf * NBUF) == 0
    per_w = r // NW
    n_chunks = per_w // CHUNK
    n_super = per_w // rows_per_buf
    n_bodies = n_super // NBUF
    idx3 = idx.reshape(NW, n_chunks, CHUNK)
    mesh = plsc.VectorSubcoreMesh(core_axis_name="c", subcore_axis_name="s")

    @functools.partial(
        pl.kernel,
        out_type=jax.ShapeDtypeStruct((r, D), jnp.float32),
        mesh=mesh,
        scratch_types=[
            pltpu.VMEM((n_chunks, CHUNK), jnp.int32),
            [pltpu.VMEM((rows_per_buf, D), jnp.float32) for _ in range(NBUF)],
            [pltpu.SemaphoreType.DMA for _ in range(NBUF)],
            [pltpu.SemaphoreType.DMA for _ in range(NBUF)],
        ],
        compiler_params=pltpu.CompilerParams(use_tc_tiling_on_sc=False),
    )
    def gather_kernel(table_hbm, idx_hbm, out_hbm, idx_v, bufs, gsems, wsems):
        wid = lax.axis_index("s") * NC + lax.axis_index("c")
        base = wid * per_w
        pltpu.sync_copy(idx_hbm.at[wid], idx_v)

        def fire_gathers(s, i):
            for u in range(SUPER):
                pltpu.make_async_copy(
                    table_hbm.at[idx_v.at[s * SUPER + u]],
                    bufs[i].at[pl.ds(u * CHUNK, CHUNK)],
                    gsems[i],
                ).start()

        def wait_gathers(s, i):
            for u in range(SUPER):
                pltpu.make_async_copy(
                    table_hbm.at[idx_v.at[s * SUPER + u]],
                    bufs[i].at[pl.ds(u * CHUNK, CHUNK)],
                    gsems[i],
                ).wait()

        def write_copy(s, i):
            return pltpu.make_async_copy(
                bufs[i],
                out_hbm.at[pl.ds(base + s * rows_per_buf, rows_per_buf)],
                wsems[i],
            )

        for i in range(NBUF):
            fire_gathers(i, i)

        def body(t, carry):
            s0 = t * NBUF
            for i in range(NBUF):
                wait_gathers(s0 + i, i)
                write_copy(s0 + i, i).start()
            for i in range(NBUF):
                write_copy(s0 + i, i).wait()

                @pl.when(t + 1 < n_bodies)
                def _():
                    fire_gathers(s0 + NBUF + i, i)

            return carry

        lax.fori_loop(0, n_bodies, body, 0)

    return gather_kernel(table, idx3)


TBLK = 4096


def _tc_transpose_table(emb_t, vocab):
    grid = pl.cdiv(vocab, 2 * TBLK)
    last = pl.cdiv(vocab, TBLK) - 1

    def body(x1_ref, x2_ref, o_ref):
        o_ref[...] = jnp.concatenate(
            [x1_ref[...].T, x2_ref[...].T], axis=1
        )

    return pl.pallas_call(
        body,
        grid=(grid,),
        in_specs=[
            pl.BlockSpec((D, TBLK), lambda i: (0, i)),
            pl.BlockSpec((D, TBLK), lambda i: (0, jnp.minimum(i + grid, last))),
        ],
        out_specs=pl.BlockSpec((TBLK, 2 * D), lambda i: (i, 0)),
        out_shape=jax.ShapeDtypeStruct((grid * TBLK, 2 * D), jnp.float32),
    )(emb_t, emb_t)


def _tc_project(yp, w2, role2, b2):
    kk, m, bb2, d2 = yp.shape
    mblk = 4
    assert m % mblk == 0

    def body(y_ref, w_ref, r_ref, b_ref, o_ref):
        c = b_ref[...]
        for k in range(kk):
            c = c + lax.dot_general(
                r_ref[k], w_ref[k], (((1,), (0,)), ((), ())),
                preferred_element_type=jnp.float32,
            )
        for j in range(mblk):
            o = jnp.broadcast_to(c, (bb2, d2))
            for k in range(kk):
                o = o + lax.dot_general(
                    y_ref[k, j], w_ref[k], (((1,), (0,)), ((), ())),
                    preferred_element_type=jnp.float32,
                )
            o_ref[j] = o

    return pl.pallas_call(
        body,
        grid=(m // mblk,),
        in_specs=[
            pl.BlockSpec((kk, mblk, bb2, d2), lambda i: (0, i, 0, 0)),
            pl.BlockSpec((kk, d2, d2), lambda i: (0, 0, 0)),
            pl.BlockSpec((kk, 1, d2), lambda i: (0, 0, 0)),
            pl.BlockSpec((1, d2), lambda i: (0, 0)),
        ],
        out_specs=pl.BlockSpec((mblk, bb2, d2), lambda i: (i, 0, 0)),
        out_shape=jax.ShapeDtypeStruct((m, bb2, d2), jnp.float32),
    )(yp, w2, role2, b2)


def kernel(x, emb_table, role_table, W, b):
    bb, m, kk = x.shape
    vocab = emb_table.shape[0]
    s = TBLK * ((vocab + 2 * TBLK - 1) // (2 * TBLK))
    xi = x.transpose(2, 1, 0).reshape(-1).astype(jnp.int32)
    idx_flat = jnp.where(xi < s, 2 * xi, 2 * (xi - s) + 1)
    t128 = _tc_transpose_table(emb_table.T, vocab)
    table_lin = t128.reshape(2 * t128.shape[0], D)
    rows = _sc_gather(table_lin, idx_flat)
    yp = rows.reshape(kk, m, bb // 2, 2 * D)
    w3 = W.reshape(kk, D, D)
    z = jnp.zeros((kk, D, D), jnp.float32)
    w2 = jnp.concatenate(
        [
            jnp.concatenate([w3, z], axis=2),
            jnp.concatenate([z, w3], axis=2),
        ],
        axis=1,
    )
    role2 = jnp.tile(role_table.reshape(kk, 1, D), (1, 1, 2))
    b2 = jnp.tile(b.reshape(1, D), (1, 2))
    out_p = _tc_project(yp, w2, role2, b2)
    return out_p.reshape(m, bb, D).transpose(1, 0, 2)


# speedup vs baseline: 2.6848x; 1.0589x over previous
import functools

import jax
import jax.numpy as jnp
from jax import lax
from jax.experimental import pallas as pl
from jax.experimental.pallas import tpu as pltpu
from jax.experimental.pallas import tpu_sc as plsc

VOCAB = 1000000
D = 64
NC = 2
NS = 16
NW = NC * NS
CHUNK = 128


SUPER = 2
NBUF = 5


def _sc_gather(table, idx):
    r = idx.shape[0]
    rows_per_buf = SUPER * CHUNK
    assert r % (NW * rows_per_buf * NBUF) == 0
    per_w = r // NW
    n_chunks = per_w // CHUNK
    n_super = per_w // rows_per_buf
    n_bodies = n_super // NBUF
    idx3 = idx.reshape(NW, n_chunks, CHUNK)
    mesh = plsc.VectorSubcoreMesh(core_axis_name="c", subcore_axis_name="s")

    @functools.partial(
        pl.kernel,
        out_type=jax.ShapeDtypeStruct((r, D), jnp.float32),
        mesh=mesh,
        scratch_types=[
            pltpu.VMEM((n_chunks, CHUNK), jnp.int32),
            [pltpu.VMEM((rows_per_buf, D), jnp.float32) for _ in range(NBUF)],
            [pltpu.SemaphoreType.DMA for _ in range(NBUF)],
            [pltpu.SemaphoreType.DMA for _ in range(NBUF)],
        ],
        compiler_params=pltpu.CompilerParams(use_tc_tiling_on_sc=False),
    )
    def gather_kernel(table_hbm, idx_hbm, out_hbm, idx_v, bufs, gsems, wsems):
        wid = lax.axis_index("s") * NC + lax.axis_index("c")
        base = wid * per_w
        pltpu.sync_copy(idx_hbm.at[wid], idx_v)

        def fire_gathers(s, i):
            for u in range(SUPER):
                pltpu.make_async_copy(
                    table_hbm.at[idx_v.at[s * SUPER + u]],
                    bufs[i].at[pl.ds(u * CHUNK, CHUNK)],
                    gsems[i],
                ).start()

        def wait_gathers(s, i):
            for u in range(SUPER):
                pltpu.make_async_copy(
                    table_hbm.at[idx_v.at[s * SUPER + u]],
                    bufs[i].at[pl.ds(u * CHUNK, CHUNK)],
                    gsems[i],
                ).wait()

        def write_copy(s, i):
            return pltpu.make_async_copy(
                bufs[i],
                out_hbm.at[pl.ds(base + s * rows_per_buf, rows_per_buf)],
                wsems[i],
            )

        for i in range(NBUF):
            fire_gathers(i, i)

        def body(t, carry):
            s0 = t * NBUF
            for i in range(NBUF):
                wait_gathers(s0 + i, i)
                write_copy(s0 + i, i).start()
            for i in range(NBUF):
                write_copy(s0 + i, i).wait()

                @pl.when(t + 1 < n_bodies)
                def _():
                    fire_gathers(s0 + NBUF + i, i)

            return carry

        lax.fori_loop(0, n_bodies, body, 0)

    return gather_kernel(table, idx3)


TBLK = 8192


def _tc_transpose_table(emb_t, vocab):
    grid = pl.cdiv(vocab, 2 * TBLK)
    last = pl.cdiv(vocab, TBLK) - 1

    def body(x1_ref, x2_ref, o_ref):
        o_ref[...] = jnp.concatenate(
            [x1_ref[...].T, x2_ref[...].T], axis=1
        )

    return pl.pallas_call(
        body,
        grid=(grid,),
        in_specs=[
            pl.BlockSpec((D, TBLK), lambda i: (0, i)),
            pl.BlockSpec((D, TBLK), lambda i: (0, jnp.minimum(i + grid, last))),
        ],
        out_specs=pl.BlockSpec((TBLK, 2 * D), lambda i: (i, 0)),
        out_shape=jax.ShapeDtypeStruct((grid * TBLK, 2 * D), jnp.float32),
    )(emb_t, emb_t)


def _tc_project(yp, w2, role2, b2):
    kk, m, bb2, d2 = yp.shape
    mblk = 4
    assert m % mblk == 0

    def body(y_ref, w_ref, r_ref, b_ref, o_ref):
        c = b_ref[...]
        for k in range(kk):
            c = c + lax.dot_general(
                r_ref[k], w_ref[k], (((1,), (0,)), ((), ())),
                preferred_element_type=jnp.float32,
            )
        for j in range(mblk):
            o = jnp.broadcast_to(c, (bb2, d2))
            for k in range(kk):
                o = o + lax.dot_general(
                    y_ref[k, j], w_ref[k], (((1,), (0,)), ((), ())),
                    preferred_element_type=jnp.float32,
                )
            o_ref[j] = o

    return pl.pallas_call(
        body,
        grid=(m // mblk,),
        in_specs=[
            pl.BlockSpec((kk, mblk, bb2, d2), lambda i: (0, i, 0, 0)),
            pl.BlockSpec((kk, d2, d2), lambda i: (0, 0, 0)),
            pl.BlockSpec((kk, 1, d2), lambda i: (0, 0, 0)),
            pl.BlockSpec((1, d2), lambda i: (0, 0)),
        ],
        out_specs=pl.BlockSpec((mblk, bb2, d2), lambda i: (i, 0, 0)),
        out_shape=jax.ShapeDtypeStruct((m, bb2, d2), jnp.float32),
    )(yp, w2, role2, b2)


def kernel(x, emb_table, role_table, W, b):
    bb, m, kk = x.shape
    vocab = emb_table.shape[0]
    s = TBLK * ((vocab + 2 * TBLK - 1) // (2 * TBLK))
    xi = x.transpose(2, 1, 0).reshape(-1).astype(jnp.int32)
    idx_flat = jnp.where(xi < s, 2 * xi, 2 * (xi - s) + 1)
    t128 = _tc_transpose_table(emb_table.T, vocab)
    table_lin = t128.reshape(2 * t128.shape[0], D)
    rows = _sc_gather(table_lin, idx_flat)
    yp = rows.reshape(kk, m, bb // 2, 2 * D)
    w3 = W.reshape(kk, D, D)
    z = jnp.zeros((kk, D, D), jnp.float32)
    w2 = jnp.concatenate(
        [
            jnp.concatenate([w3, z], axis=2),
            jnp.concatenate([z, w3], axis=2),
        ],
        axis=1,
    )
    role2 = jnp.tile(role_table.reshape(kk, 1, D), (1, 1, 2))
    b2 = jnp.tile(b.reshape(1, D), (1, 2))
    out_p = _tc_project(yp, w2, role2, b2)
    return out_p.reshape(m, bb, D).transpose(1, 0, 2)


# speedup vs baseline: 2.8170x; 1.0492x over previous
import functools

import jax
import jax.numpy as jnp
from jax import lax
from jax.experimental import pallas as pl
from jax.experimental.pallas import tpu as pltpu
from jax.experimental.pallas import tpu_sc as plsc

VOCAB = 1000000
D = 64
NC = 2
NS = 16
NW = NC * NS
CHUNK = 128


SUPER = 2
NBUF = 5


def _sc_gather(table, idx):
    r = idx.shape[0]
    rows_per_buf = SUPER * CHUNK
    assert r % (NW * rows_per_buf * NBUF) == 0
    per_w = r // NW
    n_chunks = per_w // CHUNK
    n_super = per_w // rows_per_buf
    n_bodies = n_super // NBUF
    idx3 = idx.reshape(NW, n_chunks, CHUNK)
    mesh = plsc.VectorSubcoreMesh(core_axis_name="c", subcore_axis_name="s")

    @functools.partial(
        pl.kernel,
        out_type=jax.ShapeDtypeStruct((r, D), jnp.float32),
        mesh=mesh,
        scratch_types=[
            pltpu.VMEM((n_chunks, CHUNK), jnp.int32),
            [pltpu.VMEM((rows_per_buf, D), jnp.float32) for _ in range(NBUF)],
            [pltpu.SemaphoreType.DMA for _ in range(NBUF)],
            [pltpu.SemaphoreType.DMA for _ in range(NBUF)],
        ],
        compiler_params=pltpu.CompilerParams(use_tc_tiling_on_sc=False),
    )
    def gather_kernel(table_hbm, idx_hbm, out_hbm, idx_v, bufs, gsems, wsems):
        wid = lax.axis_index("s") * NC + lax.axis_index("c")
        base = wid * per_w
        pltpu.sync_copy(idx_hbm.at[wid], idx_v)

        def fire_gathers(s, i):
            for u in range(SUPER):
                pltpu.make_async_copy(
                    table_hbm.at[idx_v.at[s * SUPER + u]],
                    bufs[i].at[pl.ds(u * CHUNK, CHUNK)],
                    gsems[i],
                ).start()

        def wait_gathers(s, i):
            for u in range(SUPER):
                pltpu.make_async_copy(
                    table_hbm.at[idx_v.at[s * SUPER + u]],
                    bufs[i].at[pl.ds(u * CHUNK, CHUNK)],
                    gsems[i],
                ).wait()

        def write_copy(s, i):
            return pltpu.make_async_copy(
                bufs[i],
                out_hbm.at[pl.ds(base + s * rows_per_buf, rows_per_buf)],
                wsems[i],
            )

        for i in range(NBUF):
            fire_gathers(i, i)

        def body(t, carry):
            s0 = t * NBUF
            for i in range(NBUF):
                wait_gathers(s0 + i, i)
                write_copy(s0 + i, i).start()
            for i in range(NBUF):
                write_copy(s0 + i, i).wait()

                @pl.when(t + 1 < n_bodies)
                def _():
                    fire_gathers(s0 + NBUF + i, i)

            return carry

        lax.fori_loop(0, n_bodies, body, 0)

    return gather_kernel(table, idx3)


TBLK = 16384


def _tc_transpose_table(emb_t, vocab):
    grid = pl.cdiv(vocab, 2 * TBLK)
    last = pl.cdiv(vocab, TBLK) - 1

    def body(x1_ref, x2_ref, o_ref):
        o_ref[...] = jnp.concatenate(
            [x1_ref[...].T, x2_ref[...].T], axis=1
        )

    return pl.pallas_call(
        body,
        grid=(grid,),
        in_specs=[
            pl.BlockSpec((D, TBLK), lambda i: (0, i)),
            pl.BlockSpec((D, TBLK), lambda i: (0, jnp.minimum(i + grid, last))),
        ],
        out_specs=pl.BlockSpec((TBLK, 2 * D), lambda i: (i, 0)),
        out_shape=jax.ShapeDtypeStruct((grid * TBLK, 2 * D), jnp.float32),
    )(emb_t, emb_t)


def _tc_project(yp, w2, role2, b2):
    kk, m, bb2, d2 = yp.shape
    mblk = 8
    assert m % mblk == 0

    def body(y_ref, w_ref, r_ref, b_ref, o_ref):
        c = b_ref[...]
        for k in range(kk):
            c = c + lax.dot_general(
                r_ref[k], w_ref[k], (((1,), (0,)), ((), ())),
                preferred_element_type=jnp.float32,
            )
        for j in range(mblk):
            o = jnp.broadcast_to(c, (bb2, d2))
            for k in range(kk):
                o = o + lax.dot_general(
                    y_ref[k, j], w_ref[k], (((1,), (0,)), ((), ())),
                    preferred_element_type=jnp.float32,
                )
            o_ref[j] = o

    return pl.pallas_call(
        body,
        grid=(m // mblk,),
        in_specs=[
            pl.BlockSpec((kk, mblk, bb2, d2), lambda i: (0, i, 0, 0)),
            pl.BlockSpec((kk, d2, d2), lambda i: (0, 0, 0)),
            pl.BlockSpec((kk, 1, d2), lambda i: (0, 0, 0)),
            pl.BlockSpec((1, d2), lambda i: (0, 0)),
        ],
        out_specs=pl.BlockSpec((mblk, bb2, d2), lambda i: (i, 0, 0)),
        out_shape=jax.ShapeDtypeStruct((m, bb2, d2), jnp.float32),
    )(yp, w2, role2, b2)


def kernel(x, emb_table, role_table, W, b):
    bb, m, kk = x.shape
    vocab = emb_table.shape[0]
    s = TBLK * ((vocab + 2 * TBLK - 1) // (2 * TBLK))
    xi = x.transpose(2, 1, 0).reshape(-1).astype(jnp.int32)
    idx_flat = jnp.where(xi < s, 2 * xi, 2 * (xi - s) + 1)
    t128 = _tc_transpose_table(emb_table.T, vocab)
    table_lin = t128.reshape(2 * t128.shape[0], D)
    rows = _sc_gather(table_lin, idx_flat)
    yp = rows.reshape(kk, m, bb // 2, 2 * D)
    w3 = W.reshape(kk, D, D)
    z = jnp.zeros((kk, D, D), jnp.float32)
    w2 = jnp.concatenate(
        [
            jnp.concatenate([w3, z], axis=2),
            jnp.concatenate([z, w3], axis=2),
        ],
        axis=1,
    )
    role2 = jnp.tile(role_table.reshape(kk, 1, D), (1, 1, 2))
    b2 = jnp.tile(b.reshape(1, D), (1, 2))
    out_p = _tc_project(yp, w2, role2, b2)
    return out_p.reshape(m, bb, D).transpose(1, 0, 2)
